# baseline (speedup 1.0000x reference)
import functools

import jax
import jax.numpy as jnp
from jax import lax
from jax.experimental import pallas as pl
from jax.experimental.pallas import tpu as pltpu
from jax.experimental.pallas import tpu_sc as plsc

N_NODES = 10000
N_EDGES = 320000
D = 128

NC = 2
NS = 16
NW = NC * NS

EDGES_PER_WORKER = N_EDGES // NW
CHUNK = 80
CHUNKS = EDGES_PER_WORKER // CHUNK
ROWS_MAIN = 640
ROWS_LAST = N_NODES - 15 * ROWS_MAIN


def _sc_scatter_body(src_hbm, dst_hbm, feat_hbm, part_hbm,
                     isrc0, isrc1, isrc2, isrc3, idst0, idst1, idst2, idst3,
                     rows0, rows1, acc,
                     sem_i0, sem_i1, sem_i2, sem_i3,
                     sem_g0, sem_g1, sem_s0, sem_s1):
    c = lax.axis_index("c")
    s = lax.axis_index("s")
    wid = s * NC + c

    @pl.when(s < 15)
    def _():
        pltpu.sync_copy(feat_hbm.at[pl.ds(s * ROWS_MAIN, ROWS_MAIN)],
                        acc.at[pl.ds(s * ROWS_MAIN, ROWS_MAIN)])

    @pl.when(s == 15)
    def _():
        pltpu.sync_copy(feat_hbm.at[pl.ds(15 * ROWS_MAIN, ROWS_LAST)],
                        acc.at[pl.ds(15 * ROWS_MAIN, ROWS_LAST)])

    plsc.subcore_barrier()

    base = wid * EDGES_PER_WORKER
    isrc = (isrc0, isrc1, isrc2, isrc3)
    idst = (idst0, idst1, idst2, idst3)
    sem_i = (sem_i0, sem_i1, sem_i2, sem_i3)
    rows = (rows0, rows1)
    sem_g = (sem_g0, sem_g1)
    sem_s = (sem_s0, sem_s1)

    def start_idx(i, j4):
        off = base + i * CHUNK
        pltpu.async_copy(src_hbm.at[pl.ds(off, CHUNK)], isrc[j4], sem_i[j4])
        pltpu.async_copy(dst_hbm.at[pl.ds(off, CHUNK)], idst[j4], sem_i[j4])

    def wait_idx(j4):
        pltpu.make_async_copy(src_hbm.at[pl.ds(0, CHUNK)], isrc[j4],
                              sem_i[j4]).wait()
        pltpu.make_async_copy(dst_hbm.at[pl.ds(0, CHUNK)], idst[j4],
                              sem_i[j4]).wait()

    def start_gather(j4, j2):
        pltpu.async_copy(feat_hbm.at[isrc[j4]], rows[j2], sem_g[j2])

    def wait_gather(j4, j2):
        pltpu.make_async_copy(feat_hbm.at[isrc[j4]], rows[j2],
                              sem_g[j2]).wait()

    def start_scatter(j4, j2):
        pltpu.async_copy(rows[j2], acc.at[idst[j4]], sem_s[j2], add=True)

    def wait_scatter(j4, j2):
        pltpu.make_async_copy(rows[j2], acc.at[idst[j4]], sem_s[j2]).wait()

    def step(i, r):
        @pl.when(i >= 1)
        def _():
            wait_scatter((r + 3) % 4, (r + 1) % 2)

        @pl.when(i + 3 < CHUNKS)
        def _():
            start_idx(i + 3, (r + 3) % 4)

        wait_gather(r, r % 2)
        start_scatter(r, r % 2)

        @pl.when(i + 1 < CHUNKS)
        def _():
            wait_idx((r + 1) % 4)
            start_gather((r + 1) % 4, (r + 1) % 2)

    start_idx(0, 0)
    start_idx(1, 1)
    start_idx(2, 2)
    wait_idx(0)
    start_gather(0, 0)

    def body(i, carry):
        for r in range(4):
            @pl.when(i % 4 == r)
            def _(r=r):
                step(i, r)
        return carry

    lax.fori_loop(0, CHUNKS, body, 0)
    wait_scatter((CHUNKS - 1) % 4, (CHUNKS - 1) % 2)
    plsc.subcore_barrier()

    @pl.when(s < 15)
    def _():
        pltpu.sync_copy(acc.at[pl.ds(s * ROWS_MAIN, ROWS_MAIN)],
                        part_hbm.at[c, pl.ds(s * ROWS_MAIN, ROWS_MAIN)])

    @pl.when(s == 15)
    def _():
        pltpu.sync_copy(acc.at[pl.ds(15 * ROWS_MAIN, ROWS_LAST)],
                        part_hbm.at[c, pl.ds(15 * ROWS_MAIN, ROWS_LAST)])


@functools.partial(jax.jit, static_argnums=())
def _sc_scatter(src, dst, features):
    mesh = plsc.VectorSubcoreMesh(core_axis_name="c", subcore_axis_name="s")
    f = pl.kernel(
        _sc_scatter_body,
        out_type=jax.ShapeDtypeStruct((NC, N_NODES, D), jnp.float32),
        mesh=mesh,
        scratch_types=(
            [pltpu.VMEM((CHUNK,), jnp.int32)] * 8
            + [pltpu.VMEM((CHUNK, D), jnp.float32)] * 2
            + [pltpu.VMEM_SHARED((N_NODES, D), jnp.float32)]
            + [pltpu.SemaphoreType.DMA] * 8
        ),
    )
    return f(src, dst, features)


def _tc_finish_body(p_ref, f_ref, w_ref, b_ref, o_ref):
    h = p_ref[0] + p_ref[1] - f_ref[...]
    norm = jnp.sqrt(jnp.sum(h * h, axis=1, keepdims=True))
    hn = h / jnp.maximum(norm, 1e-12)
    o_ref[...] = lax.dot_general(
        hn, w_ref[...], (((1,), (1,)), ((), ())),
        preferred_element_type=jnp.float32) + b_ref[...]


def _tc_finish(parts, features, W, b2d):
    R = 1000
    grid = N_NODES // R
    return pl.pallas_call(
        _tc_finish_body,
        grid=(grid,),
        in_specs=[
            pl.BlockSpec((NC, R, D), lambda i: (0, i, 0)),
            pl.BlockSpec((R, D), lambda i: (i, 0)),
            pl.BlockSpec((D, D), lambda i: (0, 0)),
            pl.BlockSpec((1, D), lambda i: (0, 0)),
        ],
        out_specs=pl.BlockSpec((R, D), lambda i: (i, 0)),
        out_shape=jax.ShapeDtypeStruct((N_NODES, D), jnp.float32),
    )(parts, features, W, b2d)


def kernel(features, edge_index, W, b):
    src = edge_index[0].astype(jnp.int32)
    dst = edge_index[1].astype(jnp.int32)
    parts = _sc_scatter(src, dst, features)
    return _tc_finish(parts, features, W, b.reshape(1, D))


# speedup vs baseline: 1.4793x; 1.4793x over previous
import functools

import jax
import jax.numpy as jnp
from jax import lax
from jax.experimental import pallas as pl
from jax.experimental.pallas import tpu as pltpu
from jax.experimental.pallas import tpu_sc as plsc

N_NODES = 10000
N_EDGES = 320000
D = 128

NC = 2
NS = 16
NW = NC * NS

EDGES_PER_WORKER = N_EDGES // NW
CHUNK = 80
CHUNKS = EDGES_PER_WORKER // CHUNK
ROWS_MAIN = 640
ROWS_LAST = N_NODES - 15 * ROWS_MAIN


NIB = 8
NRB = 4


def _sc_scatter_body(src_hbm, dst_hbm, feat_hbm, part_hbm, *refs):
    isrc = refs[0:NIB]
    idst = refs[NIB:2 * NIB]
    rows = refs[2 * NIB:2 * NIB + NRB]
    acc = refs[2 * NIB + NRB]
    sems = refs[2 * NIB + NRB + 1:]
    sem_i = sems[0:NIB]
    sem_g = sems[NIB:NIB + NRB]
    sem_s = sems[NIB + NRB:NIB + 2 * NRB]
    c = lax.axis_index("c")
    s = lax.axis_index("s")
    wid = s * NC + c

    @pl.when(s < 15)
    def _():
        pltpu.sync_copy(feat_hbm.at[pl.ds(s * ROWS_MAIN, ROWS_MAIN)],
                        acc.at[pl.ds(s * ROWS_MAIN, ROWS_MAIN)])

    @pl.when(s == 15)
    def _():
        pltpu.sync_copy(feat_hbm.at[pl.ds(15 * ROWS_MAIN, ROWS_LAST)],
                        acc.at[pl.ds(15 * ROWS_MAIN, ROWS_LAST)])

    plsc.subcore_barrier()

    base = wid * EDGES_PER_WORKER

    def start_idx(i, ji):
        off = base + i * CHUNK
        pltpu.async_copy(src_hbm.at[pl.ds(off, CHUNK)], isrc[ji], sem_i[ji])
        pltpu.async_copy(dst_hbm.at[pl.ds(off, CHUNK)], idst[ji], sem_i[ji])

    def wait_idx(ji):
        pltpu.make_async_copy(src_hbm.at[pl.ds(0, CHUNK)], isrc[ji],
                              sem_i[ji]).wait()
        pltpu.make_async_copy(dst_hbm.at[pl.ds(0, CHUNK)], idst[ji],
                              sem_i[ji]).wait()

    def start_gather(ji, jr):
        pltpu.async_copy(feat_hbm.at[isrc[ji]], rows[jr], sem_g[jr])

    def wait_gather(ji, jr):
        pltpu.make_async_copy(feat_hbm.at[isrc[ji]], rows[jr],
                              sem_g[jr]).wait()

    def start_scatter(ji, jr):
        pltpu.async_copy(rows[jr], acc.at[idst[ji]], sem_s[jr], add=True)

    def wait_scatter(ji, jr):
        pltpu.make_async_copy(rows[jr], acc.at[idst[ji]], sem_s[jr]).wait()

    def step(i, r):
        @pl.when(i >= 1)
        def _():
            wait_scatter((r - 1) % NIB, (r - 1) % NRB)

        @pl.when(i + (NIB - 1) < CHUNKS)
        def _():
            start_idx(i + NIB - 1, (r - 1) % NIB)

        wait_gather(r, r % NRB)
        start_scatter(r, r % NRB)

        @pl.when(i + 3 < CHUNKS)
        def _():
            wait_idx((r + 3) % NIB)
            start_gather((r + 3) % NIB, (r + 3) % NRB)

    for j in range(NIB - 1):
        start_idx(j, j)
    for j in range(3):
        wait_idx(j)
        start_gather(j, j)

    def body(i, carry):
        for r in range(NIB):
            @pl.when(i % NIB == r)
            def _(r=r):
                step(i, r)
        return carry

    lax.fori_loop(0, CHUNKS, body, 0)
    wait_scatter((CHUNKS - 1) % NIB, (CHUNKS - 1) % NRB)
    plsc.subcore_barrier()

    @pl.when(s < 15)
    def _():
        pltpu.sync_copy(acc.at[pl.ds(s * ROWS_MAIN, ROWS_MAIN)],
                        part_hbm.at[c, pl.ds(s * ROWS_MAIN, ROWS_MAIN)])

    @pl.when(s == 15)
    def _():
        pltpu.sync_copy(acc.at[pl.ds(15 * ROWS_MAIN, ROWS_LAST)],
                        part_hbm.at[c, pl.ds(15 * ROWS_MAIN, ROWS_LAST)])


@functools.partial(jax.jit, static_argnums=())
def _sc_scatter(src, dst, features):
    mesh = plsc.VectorSubcoreMesh(core_axis_name="c", subcore_axis_name="s")
    f = pl.kernel(
        _sc_scatter_body,
        out_type=jax.ShapeDtypeStruct((NC, N_NODES, D), jnp.float32),
        mesh=mesh,
        scratch_types=(
            [pltpu.VMEM((CHUNK,), jnp.int32)] * (2 * NIB)
            + [pltpu.VMEM((CHUNK, D), jnp.float32)] * NRB
            + [pltpu.VMEM_SHARED((N_NODES, D), jnp.float32)]
            + [pltpu.SemaphoreType.DMA] * (NIB + 2 * NRB)
        ),
    )
    return f(src, dst, features)


def _tc_finish_body(p_ref, f_ref, w_ref, b_ref, o_ref):
    h = p_ref[0] + p_ref[1] - f_ref[...]
    norm = jnp.sqrt(jnp.sum(h * h, axis=1, keepdims=True))
    hn = h / jnp.maximum(norm, 1e-12)
    o_ref[...] = lax.dot_general(
        hn, w_ref[...], (((1,), (1,)), ((), ())),
        preferred_element_type=jnp.float32) + b_ref[...]


def _tc_finish(parts, features, W, b2d):
    R = 1000
    grid = N_NODES // R
    return pl.pallas_call(
        _tc_finish_body,
        grid=(grid,),
        in_specs=[
            pl.BlockSpec((NC, R, D), lambda i: (0, i, 0)),
            pl.BlockSpec((R, D), lambda i: (i, 0)),
            pl.BlockSpec((D, D), lambda i: (0, 0)),
            pl.BlockSpec((1, D), lambda i: (0, 0)),
        ],
        out_specs=pl.BlockSpec((R, D), lambda i: (i, 0)),
        out_shape=jax.ShapeDtypeStruct((N_NODES, D), jnp.float32),
    )(parts, features, W, b2d)


def kernel(features, edge_index, W, b):
    src = edge_index[0].astype(jnp.int32)
    dst = edge_index[1].astype(jnp.int32)
    parts = _sc_scatter(src, dst, features)
    return _tc_finish(parts, features, W, b.reshape(1, D))


# speedup vs baseline: 1.5342x; 1.0371x over previous
import functools

import jax
import jax.numpy as jnp
from jax import lax
from jax.experimental import pallas as pl
from jax.experimental.pallas import tpu as pltpu
from jax.experimental.pallas import tpu_sc as plsc

N_NODES = 10000
N_EDGES = 320000
D = 128

NC = 2
NS = 16
NW = NC * NS

EDGES_PER_WORKER = N_EDGES // NW
CHUNK = 80
CHUNKS = EDGES_PER_WORKER // CHUNK
ROWS_MAIN = 640
ROWS_LAST = N_NODES - 15 * ROWS_MAIN


NIB = 8
NRB = 4
GD = 3


def _sc_scatter_body(src_hbm, dst_hbm, feat_hbm, part_hbm, *refs):
    isrc = refs[0:NIB]
    idst = refs[NIB:2 * NIB]
    rows = refs[2 * NIB:2 * NIB + NRB]
    acc = refs[2 * NIB + NRB]
    sems = refs[2 * NIB + NRB + 1:]
    sem_i = sems[0:NIB]
    sem_g = sems[NIB:NIB + NRB]
    sem_s = sems[NIB + NRB:NIB + 2 * NRB]
    sem_init = sems[NIB + 2 * NRB]
    c = lax.axis_index("c")
    s = lax.axis_index("s")
    wid = s * NC + c

    @pl.when(s < 15)
    def _():
        pltpu.async_copy(feat_hbm.at[pl.ds(s * ROWS_MAIN, ROWS_MAIN)],
                         acc.at[pl.ds(s * ROWS_MAIN, ROWS_MAIN)], sem_init)

    @pl.when(s == 15)
    def _():
        pltpu.async_copy(feat_hbm.at[pl.ds(15 * ROWS_MAIN, ROWS_LAST)],
                         acc.at[pl.ds(15 * ROWS_MAIN, ROWS_LAST)], sem_init)

    base = wid * EDGES_PER_WORKER

    def start_idx(i, ji):
        off = base + i * CHUNK
        pltpu.async_copy(src_hbm.at[pl.ds(off, CHUNK)], isrc[ji], sem_i[ji])
        pltpu.async_copy(dst_hbm.at[pl.ds(off, CHUNK)], idst[ji], sem_i[ji])

    def wait_idx(ji):
        pltpu.make_async_copy(src_hbm.at[pl.ds(0, CHUNK)], isrc[ji],
                              sem_i[ji]).wait()
        pltpu.make_async_copy(dst_hbm.at[pl.ds(0, CHUNK)], idst[ji],
                              sem_i[ji]).wait()

    def start_gather(ji, jr):
        pltpu.async_copy(feat_hbm.at[isrc[ji]], rows[jr], sem_g[jr])

    def wait_gather(ji, jr):
        pltpu.make_async_copy(feat_hbm.at[isrc[ji]], rows[jr],
                              sem_g[jr]).wait()

    def start_scatter(ji, jr):
        pltpu.async_copy(rows[jr], acc.at[idst[ji]], sem_s[jr], add=True)

    def wait_scatter(ji, jr):
        pltpu.make_async_copy(rows[jr], acc.at[idst[ji]], sem_s[jr]).wait()

    def step(i, r):
        @pl.when(i >= 1)
        def _():
            wait_scatter((r - 1) % NIB, (r - 1) % NRB)

        @pl.when(i + (NIB - 1) < CHUNKS)
        def _():
            start_idx(i + NIB - 1, (r - 1) % NIB)

        wait_gather(r, r % NRB)
        start_scatter(r, r % NRB)

        @pl.when(i + GD < CHUNKS)
        def _():
            wait_idx((r + GD) % NIB)
            start_gather((r + GD) % NIB, (r + GD) % NRB)

    for j in range(NIB - 1):
        start_idx(j, j)
    for j in range(GD):
        wait_idx(j)
        start_gather(j, j)

    @pl.when(s < 15)
    def _():
        pltpu.make_async_copy(feat_hbm.at[pl.ds(0, ROWS_MAIN)],
                              acc.at[pl.ds(0, ROWS_MAIN)], sem_init).wait()

    @pl.when(s == 15)
    def _():
        pltpu.make_async_copy(feat_hbm.at[pl.ds(0, ROWS_LAST)],
                              acc.at[pl.ds(0, ROWS_LAST)], sem_init).wait()

    plsc.subcore_barrier()

    def body(i, carry):
        for r in range(NIB):
            @pl.when(i % NIB == r)
            def _(r=r):
                step(i, r)
        return carry

    lax.fori_loop(0, CHUNKS, body, 0)
    wait_scatter((CHUNKS - 1) % NIB, (CHUNKS - 1) % NRB)
    plsc.subcore_barrier()

    @pl.when(s < 15)
    def _():
        pltpu.sync_copy(acc.at[pl.ds(s * ROWS_MAIN, ROWS_MAIN)],
                        part_hbm.at[c, pl.ds(s * ROWS_MAIN, ROWS_MAIN)])

    @pl.when(s == 15)
    def _():
        pltpu.sync_copy(acc.at[pl.ds(15 * ROWS_MAIN, ROWS_LAST)],
                        part_hbm.at[c, pl.ds(15 * ROWS_MAIN, ROWS_LAST)])


@functools.partial(jax.jit, static_argnums=())
def _sc_scatter(src, dst, features):
    mesh = plsc.VectorSubcoreMesh(core_axis_name="c", subcore_axis_name="s")
    f = pl.kernel(
        _sc_scatter_body,
        out_type=jax.ShapeDtypeStruct((NC, N_NODES, D), jnp.float32),
        mesh=mesh,
        scratch_types=(
            [pltpu.VMEM((CHUNK,), jnp.int32)] * (2 * NIB)
            + [pltpu.VMEM((CHUNK, D), jnp.float32)] * NRB
            + [pltpu.VMEM_SHARED((N_NODES, D), jnp.float32)]
            + [pltpu.SemaphoreType.DMA] * (NIB + 2 * NRB + 1)
        ),
    )
    return f(src, dst, features)


def _tc_finish_body(p_ref, f_ref, w_ref, b_ref, o_ref):
    h = p_ref[0] + p_ref[1] - f_ref[...]
    norm = jnp.sqrt(jnp.sum(h * h, axis=1, keepdims=True))
    hn = h / jnp.maximum(norm, 1e-12)
    o_ref[...] = lax.dot_general(
        hn, w_ref[...], (((1,), (1,)), ((), ())),
        preferred_element_type=jnp.float32) + b_ref[...]


def _tc_finish(parts, features, W, b2d):
    R = 2000
    grid = N_NODES // R
    return pl.pallas_call(
        _tc_finish_body,
        grid=(grid,),
        in_specs=[
            pl.BlockSpec((NC, R, D), lambda i: (0, i, 0)),
            pl.BlockSpec((R, D), lambda i: (i, 0)),
            pl.BlockSpec((D, D), lambda i: (0, 0)),
            pl.BlockSpec((1, D), lambda i: (0, 0)),
        ],
        out_specs=pl.BlockSpec((R, D), lambda i: (i, 0)),
        out_shape=jax.ShapeDtypeStruct((N_NODES, D), jnp.float32),
    )(parts, features, W, b2d)


def kernel(features, edge_index, W, b):
    src = edge_index[0].astype(jnp.int32)
    dst = edge_index[1].astype(jnp.int32)
    parts = _sc_scatter(src, dst, features)
    return _tc_finish(parts, features, W, b.reshape(1, D))


# speedup vs baseline: 1.5584x; 1.0158x over previous
import functools

import jax
import jax.numpy as jnp
from jax import lax
from jax.experimental import pallas as pl
from jax.experimental.pallas import tpu as pltpu
from jax.experimental.pallas import tpu_sc as plsc

N_NODES = 10000
N_EDGES = 320000
D = 128

NC = 2
NS = 16
NW = NC * NS

EDGES_PER_WORKER = N_EDGES // NW
CHUNK = 80
CHUNKS = EDGES_PER_WORKER // CHUNK
ROWS_MAIN = 640
ROWS_LAST = N_NODES - 15 * ROWS_MAIN


NIB = 8
NRB = 4
GD = 3


def _sc_scatter_body(src_hbm, dst_hbm, feat_hbm, part_hbm, *refs):
    isrc = refs[0:NIB]
    idst = refs[NIB:2 * NIB]
    rows = refs[2 * NIB:2 * NIB + NRB]
    acc = refs[2 * NIB + NRB]
    sems = refs[2 * NIB + NRB + 1:]
    sem_i = sems[0:NIB]
    sem_g = sems[NIB:NIB + NRB]
    sem_s = sems[NIB + NRB:NIB + 2 * NRB]
    sem_init = sems[NIB + 2 * NRB]
    c = lax.axis_index("c")
    s = lax.axis_index("s")
    wid = s * NC + c

    @pl.when(s < 15)
    def _():
        pltpu.async_copy(feat_hbm.at[pl.ds(s * ROWS_MAIN, ROWS_MAIN)],
                         acc.at[pl.ds(s * ROWS_MAIN, ROWS_MAIN)], sem_init)

    @pl.when(s == 15)
    def _():
        pltpu.async_copy(feat_hbm.at[pl.ds(15 * ROWS_MAIN, ROWS_LAST)],
                         acc.at[pl.ds(15 * ROWS_MAIN, ROWS_LAST)], sem_init)

    base = wid * EDGES_PER_WORKER

    def start_idx(i, ji):
        off = base + i * CHUNK
        pltpu.async_copy(src_hbm.at[pl.ds(off, CHUNK)], isrc[ji], sem_i[ji])
        pltpu.async_copy(dst_hbm.at[pl.ds(off, CHUNK)], idst[ji], sem_i[ji])

    def wait_idx(ji):
        pltpu.make_async_copy(src_hbm.at[pl.ds(0, CHUNK)], isrc[ji],
                              sem_i[ji]).wait()
        pltpu.make_async_copy(dst_hbm.at[pl.ds(0, CHUNK)], idst[ji],
                              sem_i[ji]).wait()

    def start_gather(ji, jr):
        pltpu.async_copy(feat_hbm.at[isrc[ji]], rows[jr], sem_g[jr])

    def wait_gather(ji, jr):
        pltpu.make_async_copy(feat_hbm.at[isrc[ji]], rows[jr],
                              sem_g[jr]).wait()

    def start_scatter(ji, jr):
        pltpu.async_copy(rows[jr], acc.at[idst[ji]], sem_s[jr], add=True)

    def wait_scatter(ji, jr):
        pltpu.make_async_copy(rows[jr], acc.at[idst[ji]], sem_s[jr]).wait()

    def step(i, r):
        @pl.when(i >= 1)
        def _():
            wait_scatter((r - 1) % NIB, (r - 1) % NRB)

        @pl.when(i + (NIB - 1) < CHUNKS)
        def _():
            start_idx(i + NIB - 1, (r - 1) % NIB)

        wait_gather(r, r % NRB)
        start_scatter(r, r % NRB)

        @pl.when(i + GD < CHUNKS)
        def _():
            wait_idx((r + GD) % NIB)
            start_gather((r + GD) % NIB, (r + GD) % NRB)

    for j in range(NIB - 1):
        start_idx(j, j)
    for j in range(GD):
        wait_idx(j)
        start_gather(j, j)

    @pl.when(s < 15)
    def _():
        pltpu.make_async_copy(feat_hbm.at[pl.ds(0, ROWS_MAIN)],
                              acc.at[pl.ds(0, ROWS_MAIN)], sem_init).wait()

    @pl.when(s == 15)
    def _():
        pltpu.make_async_copy(feat_hbm.at[pl.ds(0, ROWS_LAST)],
                              acc.at[pl.ds(0, ROWS_LAST)], sem_init).wait()

    plsc.subcore_barrier()

    def body(i, carry):
        for r in range(NIB):
            @pl.when(i % NIB == r)
            def _(r=r):
                step(i, r)
        return carry

    lax.fori_loop(0, CHUNKS, body, 0)
    wait_scatter((CHUNKS - 1) % NIB, (CHUNKS - 1) % NRB)
    plsc.subcore_barrier()

    @pl.when(s < 15)
    def _():
        pltpu.sync_copy(acc.at[pl.ds(s * ROWS_MAIN, ROWS_MAIN)],
                        part_hbm.at[c, pl.ds(s * ROWS_MAIN, ROWS_MAIN)])

    @pl.when(s == 15)
    def _():
        pltpu.sync_copy(acc.at[pl.ds(15 * ROWS_MAIN, ROWS_LAST)],
                        part_hbm.at[c, pl.ds(15 * ROWS_MAIN, ROWS_LAST)])


@functools.partial(jax.jit, static_argnums=())
def _sc_scatter(src, dst, features):
    mesh = plsc.VectorSubcoreMesh(core_axis_name="c", subcore_axis_name="s")
    f = pl.kernel(
        _sc_scatter_body,
        out_type=jax.ShapeDtypeStruct((NC, N_NODES, D), jnp.float32),
        mesh=mesh,
        scratch_types=(
            [pltpu.VMEM((CHUNK,), jnp.int32)] * (2 * NIB)
            + [pltpu.VMEM((CHUNK, D), jnp.float32)] * NRB
            + [pltpu.VMEM_SHARED((N_NODES, D), jnp.float32)]
            + [pltpu.SemaphoreType.DMA] * (NIB + 2 * NRB + 1)
        ),
    )
    return f(src, dst, features)


def _tc_finish_body(p_ref, f_ref, w_ref, b_ref, o_ref):
    h = p_ref[0] + p_ref[1] - f_ref[...]
    norm = jnp.sqrt(jnp.sum(h * h, axis=1, keepdims=True))
    hn = h / jnp.maximum(norm, 1e-12)
    o_ref[...] = lax.dot_general(
        hn, w_ref[...], (((1,), (1,)), ((), ())),
        preferred_element_type=jnp.float32) + b_ref[...]


def _tc_finish(parts, features, W, b2d):
    R = 5000
    grid = N_NODES // R
    return pl.pallas_call(
        _tc_finish_body,
        grid=(grid,),
        in_specs=[
            pl.BlockSpec((NC, R, D), lambda i: (0, i, 0)),
            pl.BlockSpec((R, D), lambda i: (i, 0)),
            pl.BlockSpec((D, D), lambda i: (0, 0)),
            pl.BlockSpec((1, D), lambda i: (0, 0)),
        ],
        out_specs=pl.BlockSpec((R, D), lambda i: (i, 0)),
        out_shape=jax.ShapeDtypeStruct((N_NODES, D), jnp.float32),
    )(parts, features, W, b2d)


def kernel(features, edge_index, W, b):
    src = edge_index[0].astype(jnp.int32)
    dst = edge_index[1].astype(jnp.int32)
    parts = _sc_scatter(src, dst, features)
    return _tc_finish(parts, features, W, b.reshape(1, D))
